# initial kernel scaffold (unmeasured)
import jax
import jax.numpy as jnp
from jax import lax
from jax.experimental import pallas as pl
from jax.experimental.pallas import tpu as pltpu


def kernel(
    x,
):
    def body(*refs):
        pass

    out_shape = jax.ShapeDtypeStruct(..., jnp.float32)
    return pl.pallas_call(body, out_shape=out_shape)(...)


# baseline (device time: 2129022 ns/iter reference)
import jax
import jax.numpy as jnp
from jax import lax
from jax.experimental import pallas as pl
from jax.experimental.pallas import tpu as pltpu

HBM = pltpu.MemorySpace.HBM


def kernel(x):
    m, n = x.shape
    out_m = 2 * m
    h = m // 2
    T = 8
    tm = h // T

    def body(x_ref, out_ref, send_half, recv_y, recv_z, v_f32, v_bf16,
             in_sem, out_sem, own_sem, send_sems, recv_sems):
        my_x = lax.axis_index("x")
        my_y = lax.axis_index("y")
        my_z = lax.axis_index("z")
        nbr_y = (my_x, 1 - my_y, my_z)
        nbr_z = (my_x, my_y, 1 - my_z)

        barrier_sem = pltpu.get_barrier_semaphore()
        for nbr in (nbr_y, nbr_z):
            pl.semaphore_signal(barrier_sem, inc=1, device_id=nbr,
                                device_id_type=pl.DeviceIdType.MESH)
        pl.semaphore_wait(barrier_sem, 2)

        own = pltpu.make_async_copy(
            x_ref, out_ref.at[pl.ds(my_y * m, m)], own_sem)
        own.start()

        src0 = my_z * h
        for t in range(T):
            cp_in = pltpu.make_async_copy(
                x_ref.at[pl.ds(src0 + t * tm, tm)], v_f32, in_sem)
            cp_in.start()
            cp_in.wait()
            send_half[t * tm:(t + 1) * tm, :] = (
                v_f32[...].astype(jnp.bfloat16))

        rdma_y = pltpu.make_async_remote_copy(
            src_ref=send_half, dst_ref=recv_y,
            send_sem=send_sems.at[0], recv_sem=recv_sems.at[0],
            device_id=nbr_y, device_id_type=pl.DeviceIdType.MESH)
        rdma_y.start()
        rdma_y.wait()

        rdma_z = pltpu.make_async_remote_copy(
            src_ref=recv_y, dst_ref=recv_z,
            send_sem=send_sems.at[1], recv_sem=recv_sems.at[1],
            device_id=nbr_z, device_id_type=pl.DeviceIdType.MESH)
        rdma_z.start()

        other = (1 - my_y) * m
        dst_y = other + my_z * h
        dst_z = other + (1 - my_z) * h
        for t in range(T):
            v_f32[...] = recv_y[t * tm:(t + 1) * tm, :].astype(jnp.float32)
            cp_out = pltpu.make_async_copy(
                v_f32, out_ref.at[pl.ds(dst_y + t * tm, tm)], out_sem)
            cp_out.start()
            cp_out.wait()

        rdma_z.wait()
        for t in range(T):
            v_f32[...] = recv_z[t * tm:(t + 1) * tm, :].astype(jnp.float32)
            cp_out = pltpu.make_async_copy(
                v_f32, out_ref.at[pl.ds(dst_z + t * tm, tm)], out_sem)
            cp_out.start()
            cp_out.wait()

        own.wait()

        @jax.named_scope("exit_barrier")
        def _exit(second_barrier):
            for nbr in (nbr_y, nbr_z):
                pl.semaphore_signal(second_barrier, inc=1, device_id=nbr,
                                    device_id_type=pl.DeviceIdType.MESH)
            pl.semaphore_wait(second_barrier, 2)

        pl.run_scoped(_exit, second_barrier=pltpu.SemaphoreType.REGULAR)

    return pl.pallas_call(
        body,
        out_shape=jax.ShapeDtypeStruct((out_m, n), jnp.float32),
        in_specs=[pl.BlockSpec(memory_space=HBM)],
        out_specs=pl.BlockSpec(memory_space=HBM),
        scratch_shapes=[
            pltpu.VMEM((h, n), jnp.bfloat16),
            pltpu.VMEM((h, n), jnp.bfloat16),
            pltpu.VMEM((h, n), jnp.bfloat16),
            pltpu.VMEM((tm, n), jnp.float32),
            pltpu.VMEM((tm, n), jnp.bfloat16),
            pltpu.SemaphoreType.DMA,
            pltpu.SemaphoreType.DMA,
            pltpu.SemaphoreType.DMA,
            pltpu.SemaphoreType.DMA((2,)),
            pltpu.SemaphoreType.DMA((2,)),
        ],
        compiler_params=pltpu.CompilerParams(
            collective_id=0, vmem_limit_bytes=60 * 1024 * 1024),
    )(x)
